# Initial kernel scaffold; baseline (speedup 1.0000x reference)
import jax
import jax.numpy as jnp
from jax.experimental import pallas as pl


def kernel(feat_user, feat_item, W_uc, b_uc, W_ic, b_ic, src_uc, dst_uc, src_ic, dst_ic):
    raise NotImplementedError("write your pallas kernel here")


# speedup vs baseline: 6.0079x; 6.0079x over previous
import dataclasses

import jax
import jax.numpy as jnp
from jax import lax
from jax.experimental import pallas as pl
from jax.experimental.pallas import tpu as pltpu
from jax.experimental.pallas import tpu_sc as plsc

N_NODE = 10000
N_EDGE = 320000
D = 128
N_TILES = 16
EDGE_PER_TILE = N_EDGE // N_TILES
CHUNK = 128
N_CHUNK = 160
PAD = N_CHUNK * CHUNK - EDGE_PER_TILE
ROWS = 10240
RPT = ROWS // N_TILES


def _sc_body(feat_hbm, src_hbm, dst_hbm, zrow_hbm, zcnt_hbm, ones_hbm,
             sums_hbm, cntw_hbm,
             acc_sh, cnt_sh, src_v, dst_v, rows_v, ones_v, cnt_t, row_t):
    c = lax.axis_index("c")
    s = lax.axis_index("s")
    w = c * N_TILES + s

    pltpu.sync_copy(zrow_hbm, acc_sh.at[pl.ds(RPT * s, RPT)])
    pltpu.sync_copy(zcnt_hbm, cnt_sh.at[pl.ds(RPT * s, RPT)])
    pltpu.sync_copy(ones_hbm, ones_v)
    plsc.subcore_barrier()

    base_row = w * N_CHUNK

    @pl.loop(0, N_CHUNK)
    def _(j):
        pltpu.sync_copy(src_hbm.at[base_row + j], src_v)
        pltpu.sync_copy(dst_hbm.at[base_row + j], dst_v)
        pltpu.sync_copy(feat_hbm.at[src_v], rows_v)
        pltpu.sync_copy(rows_v, acc_sh.at[dst_v], add=True)
        pltpu.sync_copy(ones_v, cnt_sh.at[dst_v], add=True)

    plsc.subcore_barrier()

    base = c * ROWS + RPT * s
    pltpu.sync_copy(acc_sh.at[pl.ds(RPT * s, RPT)], sums_hbm.at[pl.ds(base, RPT)])

    pltpu.sync_copy(cnt_sh.at[pl.ds(RPT * s, RPT)], cnt_t)
    pltpu.sync_copy(zrow_hbm.at[pl.ds(0, CHUNK)], row_t)
    lane0 = jnp.zeros((16,), jnp.int32)

    @pl.loop(0, RPT // CHUNK)
    def _(k):
        @pl.loop(0, CHUNK // 16)
        def _(g):
            v16 = cnt_t[pl.ds(k * CHUNK + g * 16, 16)]
            ridx = lax.iota(jnp.int32, 16) + g * 16
            plsc.store_scatter(row_t, [ridx, lane0], v16)
        pltpu.sync_copy(row_t, cntw_hbm.at[pl.ds(base + CHUNK * k, CHUNK)])


def _tc_body(s_ref, c_ref, w_ref, b_ref, o_ref):
    sums = s_ref[0]
    cnt = c_ref[0][:, 0:1]
    mean = sums * (1.0 / jnp.maximum(cnt, 1.0))
    h = jnp.dot(mean, w_ref[0], preferred_element_type=jnp.float32)
    o_ref[0] = h + jnp.where(cnt > 0.0, 1.0, 0.0) * b_ref[0]


def _prep_idx(src, dst, src_off):
    src = src.astype(jnp.int32).reshape(N_TILES, EDGE_PER_TILE) + src_off
    dst = dst.astype(jnp.int32).reshape(N_TILES, EDGE_PER_TILE)
    spread = jnp.arange(PAD, dtype=jnp.int32) % 8
    pad_src = jnp.tile(spread[None, :], (N_TILES, 1)) + src_off
    pad_dst = jnp.tile((N_NODE + spread)[None, :], (N_TILES, 1))
    src = jnp.concatenate([src, pad_src], axis=1).reshape(-1, CHUNK)
    dst = jnp.concatenate([dst, pad_dst], axis=1).reshape(-1, CHUNK)
    return src, dst


def kernel(feat_user, feat_item, W_uc, b_uc, W_ic, b_ic,
           src_uc, dst_uc, src_ic, dst_ic):
    feats = jnp.concatenate([feat_user, feat_item], axis=0)
    src0, dst0 = _prep_idx(src_uc, dst_uc, 0)
    src1, dst1 = _prep_idx(src_ic, dst_ic, N_NODE)
    src_all = jnp.concatenate([src0, src1], axis=0)
    dst_all = jnp.concatenate([dst0, dst1], axis=0)

    zrow = jnp.zeros((RPT, D), jnp.float32)
    zcnt = jnp.zeros((RPT,), jnp.float32)
    ones = jnp.ones((CHUNK,), jnp.float32)

    mesh = plsc.VectorSubcoreMesh(core_axis_name="c", subcore_axis_name="s")
    cp = pltpu.CompilerParams()
    if "needs_layout_passes" in pltpu.CompilerParams.__dataclass_fields__:
        cp = dataclasses.replace(cp, needs_layout_passes=False)
    sc_agg = pl.kernel(
        _sc_body,
        compiler_params=cp,
        out_type=(jax.ShapeDtypeStruct((2 * ROWS, D), jnp.float32),
                  jax.ShapeDtypeStruct((2 * ROWS, D), jnp.float32)),
        mesh=mesh,
        scratch_types=[
            pltpu.VMEM_SHARED((ROWS, D), jnp.float32),
            pltpu.VMEM_SHARED((ROWS,), jnp.float32),
            pltpu.VMEM((CHUNK,), jnp.int32),
            pltpu.VMEM((CHUNK,), jnp.int32),
            pltpu.VMEM((CHUNK, D), jnp.float32),
            pltpu.VMEM((CHUNK,), jnp.float32),
            pltpu.VMEM((RPT,), jnp.float32),
            pltpu.VMEM((CHUNK, D), jnp.float32),
        ],
    )
    sums, cntw = sc_agg(feats, src_all, dst_all, zrow, zcnt, ones)

    sums3 = sums.reshape(2, ROWS, D)
    cntw3 = cntw.reshape(2, ROWS, D)
    W3 = jnp.stack([W_uc, W_ic])
    b3 = jnp.stack([b_uc, b_ic]).reshape(2, 1, D)

    h = pl.pallas_call(
        _tc_body,
        grid=(2,),
        in_specs=[
            pl.BlockSpec((1, ROWS, D), lambda r: (r, 0, 0)),
            pl.BlockSpec((1, ROWS, D), lambda r: (r, 0, 0)),
            pl.BlockSpec((1, D, D), lambda r: (r, 0, 0)),
            pl.BlockSpec((1, 1, D), lambda r: (r, 0, 0)),
        ],
        out_specs=pl.BlockSpec((1, ROWS, D), lambda r: (r, 0, 0)),
        out_shape=jax.ShapeDtypeStruct((2, ROWS, D), jnp.float32),
    )(sums3, cntw3, W3, b3)

    h_item = h[0, :N_NODE]
    h_user = h[1, :N_NODE]
    return (h_user, h_item)


# speedup vs baseline: 9.5713x; 1.5931x over previous
import dataclasses

import jax
import jax.numpy as jnp
from jax import lax
from jax.experimental import pallas as pl
from jax.experimental.pallas import tpu as pltpu
from jax.experimental.pallas import tpu_sc as plsc

N_NODE = 10000
N_EDGE = 320000
D = 128
N_TILES = 16
EDGE_PER_TILE = N_EDGE // N_TILES
CHUNK = 128
GRP = 16
SLABS = 10
N_CHUNK = SLABS * GRP
PAD = N_CHUNK * CHUNK - EDGE_PER_TILE
ROWS = 10240
RPT = ROWS // N_TILES


def _sc_body(feat_hbm, src_hbm, dst_hbm, zrow_hbm, zcnt_hbm, ones_hbm,
             sums_hbm, cntw_hbm,
             acc_sh, cnt_sh, src_sl, dst_sl, rows0, rows1, ones_v, cnt_t,
             sg0, sg1, sa0, sa1, sc_sem):
    c = lax.axis_index("c")
    s = lax.axis_index("s")
    w = c * N_TILES + s
    rows = (rows0, rows1)
    sg = (sg0, sg1)
    sa = (sa0, sa1)

    def gather_start(j, b):
        pltpu.async_copy(feat_hbm.at[src_sl.at[j]], rows[b], sg[b])

    def gather_wait(b):
        pltpu.make_async_copy(feat_hbm.at[src_sl.at[0]], rows[b], sg[b]).wait()

    def rowadd_start(j, b):
        pltpu.async_copy(rows[b], acc_sh.at[dst_sl.at[j]], sa[b], add=True)

    def rowadd_wait(b):
        pltpu.make_async_copy(rows[b], acc_sh.at[dst_sl.at[0]], sa[b]).wait()

    def cnt_start(j):
        pltpu.async_copy(ones_v, cnt_sh.at[dst_sl.at[j]], sc_sem, add=True)

    def cnt_wait():
        pltpu.make_async_copy(ones_v, cnt_sh.at[dst_sl.at[0]], sc_sem).wait()

    pltpu.sync_copy(zrow_hbm, acc_sh.at[pl.ds(RPT * s, RPT)])
    pltpu.sync_copy(zcnt_hbm, cnt_sh.at[pl.ds(RPT * s, RPT)])
    pltpu.sync_copy(ones_hbm, ones_v)
    plsc.subcore_barrier()

    @pl.loop(0, SLABS)
    def _(slab):
        @pl.when(slab > 0)
        def _():
            rowadd_wait(0)
            rowadd_wait(1)
            cnt_wait()

        base_chunk = (w * SLABS + slab) * GRP
        pltpu.sync_copy(src_hbm.at[pl.ds(base_chunk, GRP)], src_sl)
        pltpu.sync_copy(dst_hbm.at[pl.ds(base_chunk, GRP)], dst_sl)
        gather_start(0, 0)

        @pl.loop(0, GRP // 2)
        def _(jj):
            for b in range(2):
                j = jj * 2 + b
                gather_wait(b)
                rowadd_start(j, b)

                @pl.when(j > 0)
                def _():
                    cnt_wait()

                cnt_start(j)
                if b == 0:
                    @pl.when(j > 0)
                    def _():
                        rowadd_wait(1)

                    gather_start(j + 1, 1)
                else:
                    @pl.when(j < GRP - 1)
                    def _():
                        rowadd_wait(0)
                        gather_start(j + 1, 0)

    rowadd_wait(0)
    rowadd_wait(1)
    cnt_wait()
    plsc.subcore_barrier()

    base = c * ROWS + RPT * s
    pltpu.sync_copy(acc_sh.at[pl.ds(RPT * s, RPT)], sums_hbm.at[pl.ds(base, RPT)])

    row_t = rows0
    pltpu.sync_copy(cnt_sh.at[pl.ds(RPT * s, RPT)], cnt_t)
    pltpu.sync_copy(zrow_hbm.at[pl.ds(0, CHUNK)], row_t)
    lane0 = jnp.zeros((16,), jnp.int32)

    @pl.loop(0, RPT // CHUNK)
    def _(k):
        @pl.loop(0, CHUNK // 16)
        def _(g):
            v16 = cnt_t[pl.ds(k * CHUNK + g * 16, 16)]
            ridx = lax.iota(jnp.int32, 16) + g * 16
            plsc.store_scatter(row_t, [ridx, lane0], v16)
        pltpu.sync_copy(row_t, cntw_hbm.at[pl.ds(base + CHUNK * k, CHUNK)])


def _tc_body(s_ref, c_ref, w_ref, b_ref, o_ref):
    sums = s_ref[0]
    cnt = c_ref[0][:, 0:1]
    mean = sums * (1.0 / jnp.maximum(cnt, 1.0))
    h = jnp.dot(mean, w_ref[0], preferred_element_type=jnp.float32)
    o_ref[0] = h + jnp.where(cnt > 0.0, 1.0, 0.0) * b_ref[0]


def _prep_idx(src, dst, src_off):
    src = src.astype(jnp.int32).reshape(N_TILES, EDGE_PER_TILE) + src_off
    dst = dst.astype(jnp.int32).reshape(N_TILES, EDGE_PER_TILE)
    spread = jnp.arange(PAD, dtype=jnp.int32) % 8
    pad_src = jnp.tile(spread[None, :], (N_TILES, 1)) + src_off
    pad_dst = jnp.tile((N_NODE + spread)[None, :], (N_TILES, 1))
    src = jnp.concatenate([src, pad_src], axis=1).reshape(-1, CHUNK)
    dst = jnp.concatenate([dst, pad_dst], axis=1).reshape(-1, CHUNK)
    return src, dst


def kernel(feat_user, feat_item, W_uc, b_uc, W_ic, b_ic,
           src_uc, dst_uc, src_ic, dst_ic):
    feats = jnp.concatenate([feat_user, feat_item], axis=0)
    src0, dst0 = _prep_idx(src_uc, dst_uc, 0)
    src1, dst1 = _prep_idx(src_ic, dst_ic, N_NODE)
    src_all = jnp.concatenate([src0, src1], axis=0)
    dst_all = jnp.concatenate([dst0, dst1], axis=0)

    zrow = jnp.zeros((RPT, D), jnp.float32)
    zcnt = jnp.zeros((RPT,), jnp.float32)
    ones = jnp.ones((CHUNK,), jnp.float32)

    mesh = plsc.VectorSubcoreMesh(core_axis_name="c", subcore_axis_name="s")
    cp = pltpu.CompilerParams()
    if "needs_layout_passes" in pltpu.CompilerParams.__dataclass_fields__:
        cp = dataclasses.replace(cp, needs_layout_passes=False)
    sc_agg = pl.kernel(
        _sc_body,
        compiler_params=cp,
        out_type=(jax.ShapeDtypeStruct((2 * ROWS, D), jnp.float32),
                  jax.ShapeDtypeStruct((2 * ROWS, D), jnp.float32)),
        mesh=mesh,
        scratch_types=[
            pltpu.VMEM_SHARED((ROWS, D), jnp.float32),
            pltpu.VMEM_SHARED((ROWS,), jnp.float32),
            pltpu.VMEM((GRP, CHUNK), jnp.int32),
            pltpu.VMEM((GRP, CHUNK), jnp.int32),
            pltpu.VMEM((CHUNK, D), jnp.float32),
            pltpu.VMEM((CHUNK, D), jnp.float32),
            pltpu.VMEM((CHUNK,), jnp.float32),
            pltpu.VMEM((RPT,), jnp.float32),
            pltpu.SemaphoreType.DMA,
            pltpu.SemaphoreType.DMA,
            pltpu.SemaphoreType.DMA,
            pltpu.SemaphoreType.DMA,
            pltpu.SemaphoreType.DMA,
        ],
    )
    sums, cntw = sc_agg(feats, src_all, dst_all, zrow, zcnt, ones)

    sums3 = sums.reshape(2, ROWS, D)
    cntw3 = cntw.reshape(2, ROWS, D)
    W3 = jnp.stack([W_uc, W_ic])
    b3 = jnp.stack([b_uc, b_ic]).reshape(2, 1, D)

    h = pl.pallas_call(
        _tc_body,
        grid=(2,),
        in_specs=[
            pl.BlockSpec((1, ROWS, D), lambda r: (r, 0, 0)),
            pl.BlockSpec((1, ROWS, D), lambda r: (r, 0, 0)),
            pl.BlockSpec((1, D, D), lambda r: (r, 0, 0)),
            pl.BlockSpec((1, 1, D), lambda r: (r, 0, 0)),
        ],
        out_specs=pl.BlockSpec((1, ROWS, D), lambda r: (r, 0, 0)),
        out_shape=jax.ShapeDtypeStruct((2, ROWS, D), jnp.float32),
    )(sums3, cntw3, W3, b3)

    h_item = h[0, :N_NODE]
    h_user = h[1, :N_NODE]
    return (h_user, h_item)


# speedup vs baseline: 9.5719x; 1.0001x over previous
import dataclasses

import jax
import jax.numpy as jnp
from jax import lax
from jax.experimental import pallas as pl
from jax.experimental.pallas import tpu as pltpu
from jax.experimental.pallas import tpu_sc as plsc

N_NODE = 10000
N_EDGE = 320000
D = 128
N_TILES = 16
EDGE_PER_TILE = N_EDGE // N_TILES
CHUNK = 128
GRP = 16
SLABS = 10
N_CHUNK = SLABS * GRP
PAD = N_CHUNK * CHUNK - EDGE_PER_TILE
ROWS = 10240
RPT = ROWS // N_TILES


def _sc_body(feat_hbm, src_hbm, dst_hbm, zrow_hbm, zcnt_hbm, ones_hbm,
             sums_hbm, cntw_hbm,
             acc_sh, cnt_sh, src_sl, dst_sl, rows0, rows1, ones_v, cnt_t,
             sg0, sg1, sa0, sa1, sc_sem):
    c = lax.axis_index("c")
    s = lax.axis_index("s")
    w = c * N_TILES + s
    rows = (rows0, rows1)
    sg = (sg0, sg1)
    sa = (sa0, sa1)

    def gather_start(j, b):
        pltpu.async_copy(feat_hbm.at[src_sl.at[j]], rows[b], sg[b])

    def gather_wait(b):
        pltpu.make_async_copy(feat_hbm.at[src_sl.at[0]], rows[b], sg[b]).wait()

    def rowadd_start(j, b):
        pltpu.async_copy(rows[b], acc_sh.at[dst_sl.at[j]], sa[b], add=True)

    def rowadd_wait(b):
        pltpu.make_async_copy(rows[b], acc_sh.at[dst_sl.at[0]], sa[b]).wait()

    def cnt_start(j):
        pltpu.async_copy(ones_v, cnt_sh.at[dst_sl.at[j]], sc_sem, add=True)

    def cnt_wait():
        pltpu.make_async_copy(ones_v, cnt_sh.at[dst_sl.at[0]], sc_sem).wait()

    pltpu.sync_copy(zrow_hbm, acc_sh.at[pl.ds(RPT * s, RPT)])
    pltpu.sync_copy(zcnt_hbm, cnt_sh.at[pl.ds(RPT * s, RPT)])
    pltpu.sync_copy(ones_hbm, ones_v)
    plsc.subcore_barrier()

    @pl.loop(0, SLABS)
    def _(slab):
        @pl.when(slab > 0)
        def _():
            rowadd_wait(0)
            rowadd_wait(1)

            @pl.loop(0, GRP)
            def _(_j):
                cnt_wait()

        base_chunk = (w * SLABS + slab) * GRP
        pltpu.sync_copy(src_hbm.at[pl.ds(base_chunk, GRP)], src_sl)
        pltpu.sync_copy(dst_hbm.at[pl.ds(base_chunk, GRP)], dst_sl)
        gather_start(0, 0)

        @pl.loop(0, GRP // 2)
        def _(jj):
            for b in range(2):
                j = jj * 2 + b
                gather_wait(b)
                rowadd_start(j, b)
                cnt_start(j)
                if b == 0:
                    @pl.when(j > 0)
                    def _():
                        rowadd_wait(1)

                    gather_start(j + 1, 1)
                else:
                    @pl.when(j < GRP - 1)
                    def _():
                        rowadd_wait(0)
                        gather_start(j + 1, 0)

    rowadd_wait(0)
    rowadd_wait(1)

    @pl.loop(0, GRP)
    def _(_j):
        cnt_wait()

    plsc.subcore_barrier()

    base = c * ROWS + RPT * s
    pltpu.sync_copy(acc_sh.at[pl.ds(RPT * s, RPT)], sums_hbm.at[pl.ds(base, RPT)])

    row_t = rows0
    pltpu.sync_copy(cnt_sh.at[pl.ds(RPT * s, RPT)], cnt_t)
    pltpu.sync_copy(zrow_hbm.at[pl.ds(0, CHUNK)], row_t)
    lane0 = jnp.zeros((16,), jnp.int32)

    @pl.loop(0, RPT // CHUNK)
    def _(k):
        @pl.loop(0, CHUNK // 16)
        def _(g):
            v16 = cnt_t[pl.ds(k * CHUNK + g * 16, 16)]
            ridx = lax.iota(jnp.int32, 16) + g * 16
            plsc.store_scatter(row_t, [ridx, lane0], v16)
        pltpu.sync_copy(row_t, cntw_hbm.at[pl.ds(base + CHUNK * k, CHUNK)])


def _tc_body(s_ref, c_ref, w_ref, b_ref, o_ref):
    sums = s_ref[0]
    cnt = c_ref[0][:, 0:1]
    mean = sums * (1.0 / jnp.maximum(cnt, 1.0))
    h = jnp.dot(mean, w_ref[0], preferred_element_type=jnp.float32)
    o_ref[0] = h + jnp.where(cnt > 0.0, 1.0, 0.0) * b_ref[0]


def _prep_idx(src, dst, src_off):
    src = src.astype(jnp.int32).reshape(N_TILES, EDGE_PER_TILE) + src_off
    dst = dst.astype(jnp.int32).reshape(N_TILES, EDGE_PER_TILE)
    spread = jnp.arange(PAD, dtype=jnp.int32) % 8
    pad_src = jnp.tile(spread[None, :], (N_TILES, 1)) + src_off
    pad_dst = jnp.tile((N_NODE + spread)[None, :], (N_TILES, 1))
    src = jnp.concatenate([src, pad_src], axis=1).reshape(-1, CHUNK)
    dst = jnp.concatenate([dst, pad_dst], axis=1).reshape(-1, CHUNK)
    return src, dst


def kernel(feat_user, feat_item, W_uc, b_uc, W_ic, b_ic,
           src_uc, dst_uc, src_ic, dst_ic):
    feats = jnp.concatenate([feat_user, feat_item], axis=0)
    src0, dst0 = _prep_idx(src_uc, dst_uc, 0)
    src1, dst1 = _prep_idx(src_ic, dst_ic, N_NODE)
    src_all = jnp.concatenate([src0, src1], axis=0)
    dst_all = jnp.concatenate([dst0, dst1], axis=0)

    zrow = jnp.zeros((RPT, D), jnp.float32)
    zcnt = jnp.zeros((RPT,), jnp.float32)
    ones = jnp.ones((CHUNK,), jnp.float32)

    mesh = plsc.VectorSubcoreMesh(core_axis_name="c", subcore_axis_name="s")
    cp = pltpu.CompilerParams()
    if "needs_layout_passes" in pltpu.CompilerParams.__dataclass_fields__:
        cp = dataclasses.replace(cp, needs_layout_passes=False)
    sc_agg = pl.kernel(
        _sc_body,
        compiler_params=cp,
        out_type=(jax.ShapeDtypeStruct((2 * ROWS, D), jnp.float32),
                  jax.ShapeDtypeStruct((2 * ROWS, D), jnp.float32)),
        mesh=mesh,
        scratch_types=[
            pltpu.VMEM_SHARED((ROWS, D), jnp.float32),
            pltpu.VMEM_SHARED((ROWS,), jnp.float32),
            pltpu.VMEM((GRP, CHUNK), jnp.int32),
            pltpu.VMEM((GRP, CHUNK), jnp.int32),
            pltpu.VMEM((CHUNK, D), jnp.float32),
            pltpu.VMEM((CHUNK, D), jnp.float32),
            pltpu.VMEM((CHUNK,), jnp.float32),
            pltpu.VMEM((RPT,), jnp.float32),
            pltpu.SemaphoreType.DMA,
            pltpu.SemaphoreType.DMA,
            pltpu.SemaphoreType.DMA,
            pltpu.SemaphoreType.DMA,
            pltpu.SemaphoreType.DMA,
        ],
    )
    sums, cntw = sc_agg(feats, src_all, dst_all, zrow, zcnt, ones)

    sums3 = sums.reshape(2, ROWS, D)
    cntw3 = cntw.reshape(2, ROWS, D)
    W3 = jnp.stack([W_uc, W_ic])
    b3 = jnp.stack([b_uc, b_ic]).reshape(2, 1, D)

    h = pl.pallas_call(
        _tc_body,
        grid=(2,),
        in_specs=[
            pl.BlockSpec((1, ROWS, D), lambda r: (r, 0, 0)),
            pl.BlockSpec((1, ROWS, D), lambda r: (r, 0, 0)),
            pl.BlockSpec((1, D, D), lambda r: (r, 0, 0)),
            pl.BlockSpec((1, 1, D), lambda r: (r, 0, 0)),
        ],
        out_specs=pl.BlockSpec((1, ROWS, D), lambda r: (r, 0, 0)),
        out_shape=jax.ShapeDtypeStruct((2, ROWS, D), jnp.float32),
    )(sums3, cntw3, W3, b3)

    h_item = h[0, :N_NODE]
    h_user = h[1, :N_NODE]
    return (h_user, h_item)


# speedup vs baseline: 10.7104x; 1.1189x over previous
import dataclasses

import jax
import jax.numpy as jnp
from jax import lax
from jax.experimental import pallas as pl
from jax.experimental.pallas import tpu as pltpu
from jax.experimental.pallas import tpu_sc as plsc

N_NODE = 10000
N_EDGE = 320000
D = 128
N_TILES = 16
EDGE_PER_TILE = N_EDGE // N_TILES
CHUNK = 128
GRP = 16
SLABS = 10
N_CHUNK = SLABS * GRP
PAD = N_CHUNK * CHUNK - EDGE_PER_TILE
ROWS = 10240
RPT = ROWS // N_TILES


def _sc_body(feat_hbm, src_hbm, dst_hbm, zrow_hbm, zcnt_hbm, ones_hbm,
             sums_hbm, cntw_hbm,
             acc_sh, cnt_sh, src_sl, dst_sl, rows0, rows1, ones_v, cnt_t,
             sg0, sg1, sa0, sa1, sc_sem):
    c = lax.axis_index("c")
    s = lax.axis_index("s")
    w = c * N_TILES + s
    rows = (rows0, rows1)
    sg = (sg0, sg1)
    sa = (sa0, sa1)

    def gather_start(j, b):
        pltpu.async_copy(feat_hbm.at[src_sl.at[j]], rows[b], sg[b])

    def gather_wait(b):
        pltpu.make_async_copy(feat_hbm.at[src_sl.at[0]], rows[b], sg[b]).wait()

    def rowadd_start(j, b):
        pltpu.async_copy(rows[b], acc_sh.at[dst_sl.at[j]], sa[b], add=True)

    def rowadd_wait(b):
        pltpu.make_async_copy(rows[b], acc_sh.at[dst_sl.at[0]], sa[b]).wait()

    def cnt_start(j):
        pltpu.async_copy(ones_v, cnt_sh.at[dst_sl.at[j]], sc_sem, add=True)

    def cnt_wait():
        pltpu.make_async_copy(ones_v, cnt_sh.at[dst_sl.at[0]], sc_sem).wait()

    pltpu.sync_copy(zrow_hbm, acc_sh.at[pl.ds(RPT * s, RPT)])
    pltpu.sync_copy(zcnt_hbm, cnt_sh.at[pl.ds(RPT * s, RPT)])
    pltpu.sync_copy(ones_hbm, ones_v)
    plsc.subcore_barrier()

    @pl.loop(0, SLABS)
    def _(slab):
        @pl.when(slab > 0)
        def _():
            rowadd_wait(0)
            rowadd_wait(1)

            @pl.loop(0, GRP)
            def _(_j):
                cnt_wait()

        base_chunk = (w * SLABS + slab) * GRP
        pltpu.sync_copy(src_hbm.at[pl.ds(base_chunk, GRP)], src_sl)
        pltpu.sync_copy(dst_hbm.at[pl.ds(base_chunk, GRP)], dst_sl)
        gather_start(0, 0)
        gather_start(1, 1)

        @pl.loop(0, GRP // 2)
        def _(jj):
            for b in range(2):
                j = jj * 2 + b
                gather_wait(b)
                rowadd_start(j, b)
                cnt_start(j)

                @pl.when(j < GRP - 2)
                def _():
                    rowadd_wait(b)
                    gather_start(j + 2, b)

    rowadd_wait(0)
    rowadd_wait(1)

    @pl.loop(0, GRP)
    def _(_j):
        cnt_wait()

    plsc.subcore_barrier()

    base = c * ROWS + RPT * s
    pltpu.sync_copy(acc_sh.at[pl.ds(RPT * s, RPT)], sums_hbm.at[pl.ds(base, RPT)])

    row_t = rows0
    pltpu.sync_copy(cnt_sh.at[pl.ds(RPT * s, RPT)], cnt_t)
    pltpu.sync_copy(zrow_hbm.at[pl.ds(0, CHUNK)], row_t)
    lane0 = jnp.zeros((16,), jnp.int32)

    @pl.loop(0, RPT // CHUNK)
    def _(k):
        @pl.loop(0, CHUNK // 16)
        def _(g):
            v16 = cnt_t[pl.ds(k * CHUNK + g * 16, 16)]
            ridx = lax.iota(jnp.int32, 16) + g * 16
            plsc.store_scatter(row_t, [ridx, lane0], v16)
        pltpu.sync_copy(row_t, cntw_hbm.at[pl.ds(base + CHUNK * k, CHUNK)])


def _tc_body(s_ref, c_ref, w_ref, b_ref, o_ref):
    sums = s_ref[0]
    cnt = c_ref[0][:, 0:1]
    mean = sums * (1.0 / jnp.maximum(cnt, 1.0))
    h = jnp.dot(mean, w_ref[0], preferred_element_type=jnp.float32)
    o_ref[0] = h + jnp.where(cnt > 0.0, 1.0, 0.0) * b_ref[0]


def _prep_idx(src, dst, src_off):
    src = src.astype(jnp.int32).reshape(N_TILES, EDGE_PER_TILE) + src_off
    dst = dst.astype(jnp.int32).reshape(N_TILES, EDGE_PER_TILE)
    spread = jnp.arange(PAD, dtype=jnp.int32) % 8
    pad_src = jnp.tile(spread[None, :], (N_TILES, 1)) + src_off
    pad_dst = jnp.tile((N_NODE + spread)[None, :], (N_TILES, 1))
    src = jnp.concatenate([src, pad_src], axis=1).reshape(-1, CHUNK)
    dst = jnp.concatenate([dst, pad_dst], axis=1).reshape(-1, CHUNK)
    return src, dst


def kernel(feat_user, feat_item, W_uc, b_uc, W_ic, b_ic,
           src_uc, dst_uc, src_ic, dst_ic):
    feats = jnp.concatenate([feat_user, feat_item], axis=0)
    src0, dst0 = _prep_idx(src_uc, dst_uc, 0)
    src1, dst1 = _prep_idx(src_ic, dst_ic, N_NODE)
    src_all = jnp.concatenate([src0, src1], axis=0)
    dst_all = jnp.concatenate([dst0, dst1], axis=0)

    zrow = jnp.zeros((RPT, D), jnp.float32)
    zcnt = jnp.zeros((RPT,), jnp.float32)
    ones = jnp.ones((CHUNK,), jnp.float32)

    mesh = plsc.VectorSubcoreMesh(core_axis_name="c", subcore_axis_name="s")
    cp = pltpu.CompilerParams()
    if "needs_layout_passes" in pltpu.CompilerParams.__dataclass_fields__:
        cp = dataclasses.replace(cp, needs_layout_passes=False)
    sc_agg = pl.kernel(
        _sc_body,
        compiler_params=cp,
        out_type=(jax.ShapeDtypeStruct((2 * ROWS, D), jnp.float32),
                  jax.ShapeDtypeStruct((2 * ROWS, D), jnp.float32)),
        mesh=mesh,
        scratch_types=[
            pltpu.VMEM_SHARED((ROWS, D), jnp.float32),
            pltpu.VMEM_SHARED((ROWS,), jnp.float32),
            pltpu.VMEM((GRP, CHUNK), jnp.int32),
            pltpu.VMEM((GRP, CHUNK), jnp.int32),
            pltpu.VMEM((CHUNK, D), jnp.float32),
            pltpu.VMEM((CHUNK, D), jnp.float32),
            pltpu.VMEM((CHUNK,), jnp.float32),
            pltpu.VMEM((RPT,), jnp.float32),
            pltpu.SemaphoreType.DMA,
            pltpu.SemaphoreType.DMA,
            pltpu.SemaphoreType.DMA,
            pltpu.SemaphoreType.DMA,
            pltpu.SemaphoreType.DMA,
        ],
    )
    sums, cntw = sc_agg(feats, src_all, dst_all, zrow, zcnt, ones)

    sums3 = sums.reshape(2, ROWS, D)
    cntw3 = cntw.reshape(2, ROWS, D)
    W3 = jnp.stack([W_uc, W_ic])
    b3 = jnp.stack([b_uc, b_ic]).reshape(2, 1, D)

    h = pl.pallas_call(
        _tc_body,
        grid=(2,),
        in_specs=[
            pl.BlockSpec((1, ROWS, D), lambda r: (r, 0, 0)),
            pl.BlockSpec((1, ROWS, D), lambda r: (r, 0, 0)),
            pl.BlockSpec((1, D, D), lambda r: (r, 0, 0)),
            pl.BlockSpec((1, 1, D), lambda r: (r, 0, 0)),
        ],
        out_specs=pl.BlockSpec((1, ROWS, D), lambda r: (r, 0, 0)),
        out_shape=jax.ShapeDtypeStruct((2, ROWS, D), jnp.float32),
    )(sums3, cntw3, W3, b3)

    h_item = h[0, :N_NODE]
    h_user = h[1, :N_NODE]
    return (h_user, h_item)


# speedup vs baseline: 11.2427x; 1.0497x over previous
import dataclasses

import jax
import jax.numpy as jnp
from jax import lax
from jax.experimental import pallas as pl
from jax.experimental.pallas import tpu as pltpu
from jax.experimental.pallas import tpu_sc as plsc

N_NODE = 10000
N_EDGE = 320000
D = 128
N_TILES = 16
EDGE_PER_TILE = N_EDGE // N_TILES
CHUNK = 128
GRP = 40
SLABS = 4
N_CHUNK = SLABS * GRP
PAD = N_CHUNK * CHUNK - EDGE_PER_TILE
ROWS = 10240
RPT = ROWS // N_TILES


def _sc_body(feat_hbm, src_hbm, dst_hbm, zrow_hbm, zcnt_hbm, ones_hbm,
             sums_hbm, cntw_hbm,
             acc_sh, cnt_sh, src_sl, dst_sl, rows0, rows1, ones_v, cnt_t,
             sg0, sg1, sa0, sa1, sc_sem):
    c = lax.axis_index("c")
    s = lax.axis_index("s")
    w = c * N_TILES + s
    rows = (rows0, rows1)
    sg = (sg0, sg1)
    sa = (sa0, sa1)

    def gather_start(j, b):
        pltpu.async_copy(feat_hbm.at[src_sl.at[j]], rows[b], sg[b])

    def gather_wait(b):
        pltpu.make_async_copy(feat_hbm.at[src_sl.at[0]], rows[b], sg[b]).wait()

    def rowadd_start(j, b):
        pltpu.async_copy(rows[b], acc_sh.at[dst_sl.at[j]], sa[b], add=True)

    def rowadd_wait(b):
        pltpu.make_async_copy(rows[b], acc_sh.at[dst_sl.at[0]], sa[b]).wait()

    def cnt_start(j):
        pltpu.async_copy(ones_v, cnt_sh.at[dst_sl.at[j]], sc_sem, add=True)

    def cnt_wait():
        pltpu.make_async_copy(ones_v, cnt_sh.at[dst_sl.at[0]], sc_sem).wait()

    pltpu.sync_copy(zrow_hbm, acc_sh.at[pl.ds(RPT * s, RPT)])
    pltpu.sync_copy(zcnt_hbm, cnt_sh.at[pl.ds(RPT * s, RPT)])
    pltpu.sync_copy(ones_hbm, ones_v)
    plsc.subcore_barrier()

    @pl.loop(0, SLABS)
    def _(slab):
        @pl.when(slab > 0)
        def _():
            rowadd_wait(0)
            rowadd_wait(1)

            @pl.loop(0, GRP)
            def _(_j):
                cnt_wait()

        base_chunk = (w * SLABS + slab) * GRP
        pltpu.sync_copy(src_hbm.at[pl.ds(base_chunk, GRP)], src_sl)
        pltpu.sync_copy(dst_hbm.at[pl.ds(base_chunk, GRP)], dst_sl)
        gather_start(0, 0)
        gather_start(1, 1)

        @pl.loop(0, GRP // 2)
        def _(jj):
            for b in range(2):
                j = jj * 2 + b
                gather_wait(b)
                rowadd_start(j, b)
                cnt_start(j)

                @pl.when(j < GRP - 2)
                def _():
                    rowadd_wait(b)
                    gather_start(j + 2, b)

    rowadd_wait(0)
    rowadd_wait(1)

    @pl.loop(0, GRP)
    def _(_j):
        cnt_wait()

    plsc.subcore_barrier()

    base = c * ROWS + RPT * s
    pltpu.sync_copy(acc_sh.at[pl.ds(RPT * s, RPT)], sums_hbm.at[pl.ds(base, RPT)])

    row_t = rows0
    pltpu.sync_copy(cnt_sh.at[pl.ds(RPT * s, RPT)], cnt_t)
    pltpu.sync_copy(zrow_hbm.at[pl.ds(0, CHUNK)], row_t)
    lane0 = jnp.zeros((16,), jnp.int32)

    @pl.loop(0, RPT // CHUNK)
    def _(k):
        @pl.loop(0, CHUNK // 16)
        def _(g):
            v16 = cnt_t[pl.ds(k * CHUNK + g * 16, 16)]
            ridx = lax.iota(jnp.int32, 16) + g * 16
            plsc.store_scatter(row_t, [ridx, lane0], v16)
        pltpu.sync_copy(row_t, cntw_hbm.at[pl.ds(base + CHUNK * k, CHUNK)])


def _tc_body(s_ref, c_ref, w_ref, b_ref, o_ref):
    sums = s_ref[0]
    cnt = c_ref[0][:, 0:1]
    mean = sums * (1.0 / jnp.maximum(cnt, 1.0))
    h = jnp.dot(mean, w_ref[0], preferred_element_type=jnp.float32)
    o_ref[0] = h + jnp.where(cnt > 0.0, 1.0, 0.0) * b_ref[0]


def _prep_idx(src, dst, src_off):
    src = src.astype(jnp.int32).reshape(N_TILES, EDGE_PER_TILE) + src_off
    dst = dst.astype(jnp.int32).reshape(N_TILES, EDGE_PER_TILE)
    spread = jnp.arange(PAD, dtype=jnp.int32) % 8
    pad_src = jnp.tile(spread[None, :], (N_TILES, 1)) + src_off
    pad_dst = jnp.tile((N_NODE + spread)[None, :], (N_TILES, 1))
    src = jnp.concatenate([src, pad_src], axis=1).reshape(-1, CHUNK)
    dst = jnp.concatenate([dst, pad_dst], axis=1).reshape(-1, CHUNK)
    return src, dst


def kernel(feat_user, feat_item, W_uc, b_uc, W_ic, b_ic,
           src_uc, dst_uc, src_ic, dst_ic):
    feats = jnp.concatenate([feat_user, feat_item], axis=0)
    src0, dst0 = _prep_idx(src_uc, dst_uc, 0)
    src1, dst1 = _prep_idx(src_ic, dst_ic, N_NODE)
    src_all = jnp.concatenate([src0, src1], axis=0)
    dst_all = jnp.concatenate([dst0, dst1], axis=0)

    zrow = jnp.zeros((RPT, D), jnp.float32)
    zcnt = jnp.zeros((RPT,), jnp.float32)
    ones = jnp.ones((CHUNK,), jnp.float32)

    mesh = plsc.VectorSubcoreMesh(core_axis_name="c", subcore_axis_name="s")
    cp = pltpu.CompilerParams()
    if "needs_layout_passes" in pltpu.CompilerParams.__dataclass_fields__:
        cp = dataclasses.replace(cp, needs_layout_passes=False)
    sc_agg = pl.kernel(
        _sc_body,
        compiler_params=cp,
        out_type=(jax.ShapeDtypeStruct((2 * ROWS, D), jnp.float32),
                  jax.ShapeDtypeStruct((2 * ROWS, D), jnp.float32)),
        mesh=mesh,
        scratch_types=[
            pltpu.VMEM_SHARED((ROWS, D), jnp.float32),
            pltpu.VMEM_SHARED((ROWS,), jnp.float32),
            pltpu.VMEM((GRP, CHUNK), jnp.int32),
            pltpu.VMEM((GRP, CHUNK), jnp.int32),
            pltpu.VMEM((CHUNK, D), jnp.float32),
            pltpu.VMEM((CHUNK, D), jnp.float32),
            pltpu.VMEM((CHUNK,), jnp.float32),
            pltpu.VMEM((RPT,), jnp.float32),
            pltpu.SemaphoreType.DMA,
            pltpu.SemaphoreType.DMA,
            pltpu.SemaphoreType.DMA,
            pltpu.SemaphoreType.DMA,
            pltpu.SemaphoreType.DMA,
        ],
    )
    sums, cntw = sc_agg(feats, src_all, dst_all, zrow, zcnt, ones)

    sums3 = sums.reshape(2, ROWS, D)
    cntw3 = cntw.reshape(2, ROWS, D)
    W3 = jnp.stack([W_uc, W_ic])
    b3 = jnp.stack([b_uc, b_ic]).reshape(2, 1, D)

    h = pl.pallas_call(
        _tc_body,
        grid=(2,),
        in_specs=[
            pl.BlockSpec((1, ROWS, D), lambda r: (r, 0, 0)),
            pl.BlockSpec((1, ROWS, D), lambda r: (r, 0, 0)),
            pl.BlockSpec((1, D, D), lambda r: (r, 0, 0)),
            pl.BlockSpec((1, 1, D), lambda r: (r, 0, 0)),
        ],
        out_specs=pl.BlockSpec((1, ROWS, D), lambda r: (r, 0, 0)),
        out_shape=jax.ShapeDtypeStruct((2, ROWS, D), jnp.float32),
    )(sums3, cntw3, W3, b3)

    h_item = h[0, :N_NODE]
    h_user = h[1, :N_NODE]
    return (h_user, h_item)
